# baseline (speedup 1.0000x reference)
import functools

import jax
import jax.numpy as jnp
from jax import lax
from jax.experimental import pallas as pl
from jax.experimental.pallas import tpu as pltpu
from jax.experimental.pallas import tpu_sc as plsc

_NC = 2
_NS = 16
_NW = _NC * _NS


def _field_embedding_lookup(x, W, *, B, F, V, E):
    bc = B // _NW

    mesh = plsc.VectorSubcoreMesh(
        core_axis_name="c", subcore_axis_name="s",
        num_cores=_NC, num_subcores=_NS)

    @functools.partial(
        pl.kernel,
        out_type=jax.ShapeDtypeStruct((B, F, E), jnp.float32),
        mesh=mesh,
        scratch_types=[
            pltpu.VMEM((2, bc), jnp.int32),
            pltpu.VMEM((2, bc, E), jnp.float32),
            pltpu.SemaphoreType.DMA,
        ],
        compiler_params=pltpu.CompilerParams(use_tc_tiling_on_sc=False),
    )
    def k(xT_hbm, W_hbm, out_hbm, idx_v, rows_v, sem):
        wid = lax.axis_index("s") * _NC + lax.axis_index("c")
        b0 = wid * bc
        for f in range(F):
            p = f % 2
            pltpu.sync_copy(xT_hbm.at[f, pl.ds(b0, bc)], idx_v.at[p])
            pltpu.async_copy(
                W_hbm.at[f].at[idx_v.at[p]], rows_v.at[p], sem).wait()
            pltpu.sync_copy(rows_v.at[p], out_hbm.at[pl.ds(b0, bc), f])

    return k(x, W)


def kernel(x, W):
    B, F = x.shape
    _, V, E = W.shape
    xT = x.T
    return _field_embedding_lookup(xT, W, B=B, F=F, V=V, E=E)


# speedup vs baseline: 1.5074x; 1.5074x over previous
import functools

import jax
import jax.numpy as jnp
from jax import lax
from jax.experimental import pallas as pl
from jax.experimental.pallas import tpu as pltpu
from jax.experimental.pallas import tpu_sc as plsc

_NC = 2
_NS = 16
_NW = _NC * _NS


def _field_embedding_lookup(xTf, Wp, *, B, F, V, E):
    assert E == _NW
    n_sl = B // 16

    mesh = plsc.VectorSubcoreMesh(
        core_axis_name="c", subcore_axis_name="s",
        num_cores=_NC, num_subcores=_NS)

    @functools.partial(
        pl.kernel,
        out_type=jax.ShapeDtypeStruct((F, E, B), jnp.float32),
        mesh=mesh,
        scratch_types=[
            pltpu.VMEM((V,), jnp.float32),
            pltpu.VMEM((B,), jnp.float32),
        ],
        compiler_params=pltpu.CompilerParams(
            use_tc_tiling_on_sc=False, needs_layout_passes=False),
    )
    def k(xTf_hbm, Wp_hbm, out_hbm, plane_v, buf_v):
        e = lax.axis_index("s") * _NC + lax.axis_index("c")
        for f in range(F):
            pltpu.sync_copy(Wp_hbm.at[f, e], plane_v)
            pltpu.sync_copy(xTf_hbm.at[f], buf_v)

            def body(i, _):
                s = pl.ds(i * 16, 16)
                iv = plsc.bitcast(buf_v[s], jnp.int32)
                buf_v[s] = plsc.load_gather(plane_v, [iv])
                return ()
            lax.fori_loop(0, n_sl, body, ())

            pltpu.sync_copy(buf_v, out_hbm.at[f, e])

    return k(xTf, Wp)


def kernel(x, W):
    B, F = x.shape
    _, V, E = W.shape
    xTf = lax.bitcast_convert_type(x.T, jnp.float32)
    Wp = jnp.transpose(W, (0, 2, 1))
    out_p = _field_embedding_lookup(xTf, Wp, B=B, F=F, V=V, E=E)
    return jnp.transpose(out_p, (2, 0, 1))


# speedup vs baseline: 6.4470x; 4.2768x over previous
import functools

import jax
import jax.numpy as jnp
from jax import lax
from jax.experimental import pallas as pl
from jax.experimental.pallas import tpu as pltpu
from jax.experimental.pallas import tpu_sc as plsc

_NC = 2
_NS = 16
_NW = _NC * _NS


def _field_embedding_lookup(xTf, Wp, *, B, F, V, E):
    assert E == _NW
    n_sl = B // 16

    mesh = plsc.VectorSubcoreMesh(
        core_axis_name="c", subcore_axis_name="s",
        num_cores=_NC, num_subcores=_NS)

    @functools.partial(
        pl.kernel,
        out_type=jax.ShapeDtypeStruct((F, E, B), jnp.float32),
        mesh=mesh,
        scratch_types=[
            pltpu.VMEM((V,), jnp.float32),
            pltpu.VMEM((B,), jnp.float32),
            pltpu.SemaphoreType.DMA,
        ],
        compiler_params=pltpu.CompilerParams(
            use_tc_tiling_on_sc=True, needs_layout_passes=False),
    )
    def k(xTf_hbm, Wp_hbm, out_hbm, plane_v, buf_v, sem_w):
        e = lax.axis_index("s") * _NC + lax.axis_index("c")
        pltpu.sync_copy(Wp_hbm.at[0, e], plane_v)
        pltpu.sync_copy(xTf_hbm.at[0], buf_v)
        for f in range(F):
            @plsc.parallel_loop(0, n_sl, unroll=8)
            def body(i):
                s = pl.ds(i * 16, 16)
                iv = plsc.bitcast(buf_v[s], jnp.int32)
                buf_v[s] = plsc.load_gather(plane_v, [iv])

            wr = pltpu.async_copy(buf_v, out_hbm.at[f, e], sem_w)
            if f + 1 < F:
                pltpu.sync_copy(Wp_hbm.at[f + 1, e], plane_v)
            wr.wait()
            if f + 1 < F:
                pltpu.sync_copy(xTf_hbm.at[f + 1], buf_v)

    return k(xTf, Wp)


def kernel(x, W):
    B, F = x.shape
    _, V, E = W.shape
    xTf = lax.bitcast_convert_type(x.T, jnp.float32)
    Wp = jnp.transpose(W, (0, 2, 1))
    out_p = _field_embedding_lookup(xTf, Wp, B=B, F=F, V=V, E=E)
    return jnp.transpose(out_p, (2, 0, 1))
